# baseline (speedup 1.0000x reference)
import functools

import jax
import jax.numpy as jnp
from jax import lax
from jax.experimental import pallas as pl
from jax.experimental.pallas import tpu as pltpu
from jax.experimental.pallas import tpu_sc as plsc

D = 32
SEQ = 50
BATCH = 4096
B = BATCH * SEQ
NW = 32
B_PER_W = BATCH // NW
BC = 32
NCHUNK = B_PER_W // BC
ROWS = BC * SEQ * D // 128
OUT_ROWS = B * D // 128

_mesh = plsc.VectorSubcoreMesh(core_axis_name="c", subcore_axis_name="s")


@functools.partial(
    pl.kernel,
    mesh=_mesh,
    out_type=jax.ShapeDtypeStruct((OUT_ROWS, 128), jnp.float32),
    scratch_types=[
        pltpu.VMEM((SEQ, B_PER_W), jnp.int32),
        pltpu.VMEM((ROWS, 128), jnp.float32),
        pltpu.VMEM((ROWS, 128), jnp.float32),
        pltpu.SemaphoreType.DMA,
        pltpu.SemaphoreType.DMA,
    ],
)
def _gather_kernel(idx_hbm, table_hbm, out_hbm, idx_v, buf_a, buf_b, sem_a, sem_b):
    wid = lax.axis_index("s") * 2 + lax.axis_index("c")
    bcol = pl.multiple_of(wid * B_PER_W, B_PER_W)
    out_row = wid * (B_PER_W * SEQ * D // 128)
    pltpu.sync_copy(idx_hbm.at[:, pl.ds(bcol, B_PER_W)], idx_v)

    def enqueue_chunk(c, buf, sem):
        def s_body(s, _):
            for bg in range(BC // 16):
                vec = idx_v[s, pl.ds(c * BC + bg * 16, 16)]
                for j in range(16):
                    r = vec[j]
                    t = lax.shift_right_logical(r, 3)
                    sub = lax.bitwise_and(r, 7)
                    k = (bg * 16 + j) * SEQ + s
                    row = lax.shift_right_logical(k, 2)
                    col = lax.bitwise_and(k, 3) * 32
                    pltpu.async_copy(
                        table_hbm.at[t, sub], buf.at[row, pl.ds(col, 32)], sem
                    )
            return 0

        lax.fori_loop(0, SEQ, s_body, 0)

    def drain(buf, sem):
        pltpu.make_async_copy(out_hbm.at[pl.ds(0, ROWS)], buf, sem).wait()

    def pair_body(p, _):
        ca = p * 2
        enqueue_chunk(ca, buf_a, sem_a)
        enqueue_chunk(ca + 1, buf_b, sem_b)
        drain(buf_a, sem_a)
        oa = pl.multiple_of(out_row + ca * ROWS, 8)
        pltpu.sync_copy(buf_a, out_hbm.at[pl.ds(oa, ROWS)])
        drain(buf_b, sem_b)
        ob = pl.multiple_of(out_row + (ca + 1) * ROWS, 8)
        pltpu.sync_copy(buf_b, out_hbm.at[pl.ds(ob, ROWS)])
        return 0

    lax.fori_loop(0, NCHUNK // 2, pair_body, 0)


def kernel(input, embeddings):
    idx_t = input.T.astype(jnp.int32)
    table3 = embeddings.reshape(125000, 8, 32)
    out = _gather_kernel(idx_t, table3)
    return out.reshape(BATCH, SEQ, D)
